# baseline (speedup 1.0000x reference)
import jax
import jax.numpy as jnp
from jax import lax
from jax.experimental import pallas as pl
from jax.experimental.pallas import tpu as pltpu
from jax.experimental.pallas import tpu_sc as plsc

B = 16384
F = 26
VOCAB = 1000000
NC, NS = 2, 16
FC = F // NC
BT = B // NS
RT = BT // 128
VR = VOCAB // 8


def _sc_body(xt_hbm, tab_hbm, out_hbm, idx_v, big_v, gsem):
    c = lax.axis_index("c")
    s = lax.axis_index("s")

    pltpu.sync_copy(xt_hbm.at[pl.ds(c * FC, FC), s], idx_v)

    def shift(k, _):
        for l in range(8):
            sl = pl.ds(l * 16, 16)
            idx_v[k // 8, k % 8, sl] = idx_v[k // 8, k % 8, sl] >> 3
        return 0

    lax.fori_loop(0, FC * RT, shift, 0)

    def per_field(floc, _):
        f = c * FC + floc
        row = tab_hbm.at[f]
        for r in range(RT):
            pltpu.make_async_copy(
                row.at[idx_v.at[floc, r]],
                big_v.at[pl.ds(r * 128, 128)],
                gsem,
            ).start()
        for r in range(RT):
            pltpu.make_async_copy(
                row.at[idx_v.at[floc, r]],
                big_v.at[pl.ds(r * 128, 128)],
                gsem,
            ).wait()
        pltpu.sync_copy(big_v, out_hbm.at[c, floc, s])
        return 0

    lax.fori_loop(0, FC, per_field, 0)


def _combine_body(p_ref, x_ref, o_ref):
    f = pl.program_id(0)
    sel = x_ref[0] & 7
    t_idx = lax.rem(lax.broadcasted_iota(jnp.int32, (NS, BT * 8), 1), 8)
    masked = jnp.where(sel == t_idx, p_ref[0], 0.0)
    m = masked.reshape(NS * 64, 128)
    sgrp = (lax.broadcasted_iota(jnp.int32, (128, 16), 0) // 8
            == lax.broadcasted_iota(jnp.int32, (128, 16), 1))
    r = jnp.dot(m, sgrp.astype(jnp.float32),
                preferred_element_type=jnp.float32).reshape(NS, 64, 16)

    @pl.when(f == 0)
    def _():
        o_ref[...] = jnp.zeros_like(o_ref)

    o_ref[...] += r

    @pl.when(f == F - 1)
    def _():
        o_ref[...] = 1.0 / (1.0 + jnp.exp(-o_ref[...]))


@jax.jit
def kernel(X, linear_tables):
    xt = X.T.reshape(F, NS, RT, 128)
    tab = linear_tables.reshape(F, VR, 8)
    mesh = plsc.VectorSubcoreMesh(
        core_axis_name="c", subcore_axis_name="s",
        num_cores=NC, num_subcores=NS)
    partial = pl.kernel(
        _sc_body,
        out_type=jax.ShapeDtypeStruct((NC, FC, NS, BT, 8), jnp.float32),
        mesh=mesh,
        compiler_params=pltpu.CompilerParams(use_tc_tiling_on_sc=False),
        scratch_types=[
            pltpu.VMEM((FC, RT, 128), jnp.int32),
            pltpu.VMEM((BT, 8), jnp.float32),
            pltpu.SemaphoreType.DMA,
        ],
    )(xt, tab)
    xrep = jnp.broadcast_to(
        X.T.reshape(F, NS, BT, 1), (F, NS, BT, 8),
    ).reshape(F, NS, BT * 8)
    out = pl.pallas_call(
        _combine_body,
        grid=(F,),
        in_specs=[
            pl.BlockSpec((1, NS, BT * 8), lambda f: (f, 0, 0)),
            pl.BlockSpec((1, NS, BT * 8), lambda f: (f, 0, 0)),
        ],
        out_specs=pl.BlockSpec((NS, 64, 16), lambda f: (0, 0, 0)),
        out_shape=jax.ShapeDtypeStruct((NS, 64, 16), jnp.float32),
    )(partial.reshape(F, NS, BT * 8), xrep)
    return out.reshape(B, 1)


# speedup vs baseline: 6.3159x; 6.3159x over previous
import jax
import jax.numpy as jnp
from jax import lax
from jax.experimental import pallas as pl
from jax.experimental.pallas import tpu as pltpu
from jax.experimental.pallas import tpu_sc as plsc

B = 16384
F = 26
VOCAB = 1000000
NC, NS = 2, 16
NW = NC * NS
BW = B // NW
RT = BW // 128


def _sc_body(xt_hbm, tab_hbm, out_hbm, idx_v, vals_v, acc_v, gsem):
    wid = lax.axis_index("s") * NC + lax.axis_index("c")
    pltpu.sync_copy(xt_hbm.at[wid], idx_v)

    def per_field(f, _):
        row = tab_hbm.at[f]
        for r in range(RT):
            pltpu.make_async_copy(
                row.at[idx_v.at[f, r]], vals_v.at[r], gsem,
            ).start()
        for r in range(RT):
            pltpu.make_async_copy(
                row.at[idx_v.at[f, r]], vals_v.at[r], gsem,
            ).wait()

        @pl.when(f == 0)
        def _():
            for r in range(RT):
                for k in range(8):
                    sl = pl.ds(r * 128 + k * 16, 16)
                    acc_v[sl] = vals_v[r, pl.ds(k * 16, 16)]

        @pl.when(f > 0)
        def _():
            for r in range(RT):
                for k in range(8):
                    sl = pl.ds(r * 128 + k * 16, 16)
                    acc_v[sl] = acc_v[sl] + vals_v[r, pl.ds(k * 16, 16)]

        return 0

    lax.fori_loop(0, F, per_field, 0)

    def sig(k, _):
        sl = pl.ds(k * 16, 16)
        acc_v[sl] = 1.0 / (1.0 + jnp.exp(-acc_v[sl]))
        return 0

    lax.fori_loop(0, BW // 16, sig, 0)
    pltpu.sync_copy(acc_v, out_hbm.at[pl.ds(wid * BW, BW)])


@jax.jit
def kernel(X, linear_tables):
    xt = X.reshape(NW, BW, F).transpose(0, 2, 1).reshape(NW, F, RT, 128)
    rows = [
        lax.slice(linear_tables, (f, 0, 0), (f + 1, VOCAB, 1)).reshape(VOCAB)
        for f in range(F)
    ]
    tab = jnp.stack(rows)
    mesh = plsc.VectorSubcoreMesh(
        core_axis_name="c", subcore_axis_name="s",
        num_cores=NC, num_subcores=NS)
    out = pl.kernel(
        _sc_body,
        out_type=jax.ShapeDtypeStruct((B,), jnp.float32),
        mesh=mesh,
        compiler_params=pltpu.CompilerParams(use_tc_tiling_on_sc=False),
        scratch_types=[
            pltpu.VMEM((F, RT, 128), jnp.int32),
            pltpu.VMEM((RT, 128), jnp.float32),
            pltpu.VMEM((BW,), jnp.float32),
            pltpu.SemaphoreType.DMA,
        ],
    )(xt, tab)
    return out.reshape(B, 1)


# speedup vs baseline: 7.0670x; 1.1189x over previous
import jax
import jax.numpy as jnp
from jax import lax
from jax.experimental import pallas as pl
from jax.experimental.layout import Layout, with_layout_constraint
from jax.experimental.pallas import tpu as pltpu
from jax.experimental.pallas import tpu_sc as plsc

B = 16384
F = 26
VOCAB = 1000000
NC, NS = 2, 16
NW = NC * NS
BW = B // NW
RT = BW // 128


def _sc_body(xt_hbm, tab_hbm, out_hbm, idx_v, vals_v, acc_v, gsem):
    wid = lax.axis_index("s") * NC + lax.axis_index("c")
    pltpu.sync_copy(xt_hbm.at[wid], idx_v)

    def per_field(f, _):
        row = tab_hbm.at[f]
        for r in range(RT):
            pltpu.make_async_copy(
                row.at[idx_v.at[f, r]], vals_v.at[r], gsem,
            ).start()
        for r in range(RT):
            pltpu.make_async_copy(
                row.at[idx_v.at[f, r]], vals_v.at[r], gsem,
            ).wait()

        @pl.when(f == 0)
        def _():
            for r in range(RT):
                for k in range(8):
                    sl = pl.ds(r * 128 + k * 16, 16)
                    acc_v[sl] = vals_v[r, pl.ds(k * 16, 16)]

        @pl.when(f > 0)
        def _():
            for r in range(RT):
                for k in range(8):
                    sl = pl.ds(r * 128 + k * 16, 16)
                    acc_v[sl] = acc_v[sl] + vals_v[r, pl.ds(k * 16, 16)]

        return 0

    lax.fori_loop(0, F, per_field, 0)

    def sig(k, _):
        sl = pl.ds(k * 16, 16)
        acc_v[sl] = 1.0 / (1.0 + jnp.exp(-acc_v[sl]))
        return 0

    lax.fori_loop(0, BW // 16, sig, 0)
    pltpu.sync_copy(acc_v, out_hbm.at[pl.ds(wid * BW, BW)])


@jax.jit
def kernel(X, linear_tables):
    xt = X.reshape(NW, BW, F).transpose(0, 2, 1).reshape(NW, F, RT, 128)
    tab = jnp.squeeze(linear_tables, -1)
    tab = with_layout_constraint(
        tab, Layout(major_to_minor=(0, 1), tiling=((8, 128),)))
    tab = jax.lax.optimization_barrier(tab)
    mesh = plsc.VectorSubcoreMesh(
        core_axis_name="c", subcore_axis_name="s",
        num_cores=NC, num_subcores=NS)
    out = pl.kernel(
        _sc_body,
        out_type=jax.ShapeDtypeStruct((B,), jnp.float32),
        mesh=mesh,
        compiler_params=pltpu.CompilerParams(use_tc_tiling_on_sc=False),
        scratch_types=[
            pltpu.VMEM((F, RT, 128), jnp.int32),
            pltpu.VMEM((RT, 128), jnp.float32),
            pltpu.VMEM((BW,), jnp.float32),
            pltpu.SemaphoreType.DMA,
        ],
    )(xt, tab)
    return out.reshape(B, 1)
